# baseline (speedup 1.0000x reference)
import functools

import jax
import jax.numpy as jnp
from jax import lax
from jax.experimental import pallas as pl
from jax.experimental.pallas import tpu as pltpu
from jax.experimental.pallas import tpu_sc as plsc

_EPS = 1e-08
_M_BLK = 256
_N_BLK = 1024


def _norm_body(cb_ref, cb1_ref, cb2_ref):
    c = cb_ref[...]
    r1 = jnp.sqrt(jnp.sum(c * c, axis=1, keepdims=True))
    c1 = c / (r1 + _EPS)
    r2 = jnp.sqrt(jnp.sum(c1 * c1, axis=1, keepdims=True))
    cb1_ref[...] = c1
    cb2_ref[...] = c1 / (r2 + _EPS)


def _dist_body(num_codes, x_ref, cbn_ref, xn_ref, idx_ref, best_ref, bidx_ref):
    j = pl.program_id(1)
    xv = x_ref[...]
    r = jnp.sqrt(jnp.sum(xv * xv, axis=1, keepdims=True))
    xn = xv / (r + _EPS)

    @pl.when(j == 0)
    def _():
        xn_ref[...] = xn
        best_ref[...] = jnp.full(best_ref.shape, -jnp.inf, jnp.float32)
        bidx_ref[...] = jnp.zeros(bidx_ref.shape, jnp.int32)

    scores = lax.dot_general(xn, cbn_ref[...], (((1,), (1,)), ((), ())),
                             preferred_element_type=jnp.float32)
    m = jnp.max(scores, axis=1, keepdims=True)
    ids = lax.broadcasted_iota(jnp.int32, scores.shape, 1)
    loc = jnp.min(jnp.where(scores == m, ids, num_codes), axis=1,
                  keepdims=True)
    bv = best_ref[...]
    upd = m > bv
    best_ref[...] = jnp.where(upd, m, bv)
    bidx_ref[...] = jnp.where(upd, loc + j * _N_BLK, bidx_ref[...])

    @pl.when(j == pl.num_programs(1) - 1)
    def _():
        idx_ref[...] = bidx_ref[...]


@functools.lru_cache(maxsize=None)
def _build_tc(num_tokens, num_codes, dim):
    norm = pl.pallas_call(
        _norm_body,
        grid=(num_codes // _N_BLK,),
        in_specs=[pl.BlockSpec((_N_BLK, dim), lambda i: (i, 0))],
        out_specs=[pl.BlockSpec((_N_BLK, dim), lambda i: (i, 0)),
                   pl.BlockSpec((_N_BLK, dim), lambda i: (i, 0))],
        out_shape=[jax.ShapeDtypeStruct((num_codes, dim), jnp.float32),
                   jax.ShapeDtypeStruct((num_codes, dim), jnp.float32)],
    )
    dist = pl.pallas_call(
        functools.partial(_dist_body, num_codes),
        grid=(num_tokens // _M_BLK, num_codes // _N_BLK),
        in_specs=[pl.BlockSpec((_M_BLK, dim), lambda i, j: (i, 0)),
                  pl.BlockSpec((_N_BLK, dim), lambda i, j: (j, 0))],
        out_specs=[pl.BlockSpec((_M_BLK, dim), lambda i, j: (i, 0)),
                   pl.BlockSpec((_M_BLK, 1), lambda i, j: (i, 0))],
        out_shape=[jax.ShapeDtypeStruct((num_tokens, dim), jnp.float32),
                   jax.ShapeDtypeStruct((num_tokens, 1), jnp.int32)],
        scratch_shapes=[pltpu.VMEM((_M_BLK, 1), jnp.float32),
                        pltpu.VMEM((_M_BLK, 1), jnp.int32)],
    )
    return norm, dist


@functools.lru_cache(maxsize=None)
def _build_gather(num_tokens, num_codes, dim):
    info = plsc.get_sparse_core_info()
    nw = info.num_cores * info.num_subcores
    bpw = num_tokens // nw
    chunk = 128
    mesh = plsc.VectorSubcoreMesh(core_axis_name="c", subcore_axis_name="s")

    @functools.partial(
        pl.kernel, mesh=mesh,
        out_type=jax.ShapeDtypeStruct((num_tokens, dim), jnp.float32),
        scratch_types=[pltpu.VMEM((bpw,), jnp.int32),
                       pltpu.VMEM((bpw, dim), jnp.float32),
                       pltpu.SemaphoreType.DMA],
    )
    def gather(table_hbm, idx_hbm, out_hbm, idx_v, rows_v, sem):
        wid = lax.axis_index("s") * info.num_cores + lax.axis_index("c")
        base = wid * bpw
        pltpu.sync_copy(idx_hbm.at[pl.ds(base, bpw)], idx_v)
        copies = [
            pltpu.async_copy(table_hbm.at[idx_v.at[pl.ds(c * chunk, chunk)]],
                             rows_v.at[pl.ds(c * chunk, chunk)], sem)
            for c in range(bpw // chunk)
        ]
        for cp in copies:
            cp.wait()
        pltpu.sync_copy(rows_v, out_hbm.at[pl.ds(base, bpw)])

    return gather


def kernel(x, codebook, training):
    del training
    b, t, dim = x.shape
    num_tokens = b * t
    num_codes = codebook.shape[0]
    norm, dist = _build_tc(num_tokens, num_codes, dim)
    gather = _build_gather(num_tokens, num_codes, dim)

    cb1, cbn = norm(codebook)
    xn2, idx2 = dist(x.reshape(num_tokens, dim), cbn)
    idx = idx2.reshape(num_tokens)
    z = gather(cb1, idx).reshape(b, t, dim)
    xn = xn2.reshape(b, t, dim)
    indices = idx.reshape(b, t)
    return (z, z, xn, indices)
